# baseline (speedup 1.0000x reference)
import functools

import jax
import jax.numpy as jnp
from jax import lax
from jax.experimental import pallas as pl
from jax.experimental.pallas import tpu as pltpu
from jax.experimental.pallas import tpu_sc as plsc

NUM_CLASSES = 80
FPN_STRIDE = 8.0
NMS_PRE_TOPK = 1000
NMS_THRESHOLD = 0.6
NMS_POST_TOPK = 100
IMG_H = 800
IMG_W = 1333
N_LOCS = 20000
M = 1024
SCORE_BLK = 2000


def _make_sc_gather():
    info = plsc.get_sparse_core_info()
    nc, ns = info.num_cores, info.num_subcores
    nw = nc * ns
    bpw = M // nw
    mesh = plsc.VectorSubcoreMesh(core_axis_name="c", subcore_axis_name="s")

    @functools.partial(
        pl.kernel, mesh=mesh,
        out_type=jax.ShapeDtypeStruct((M, 128), jnp.float32),
        scratch_types=[
            pltpu.VMEM((bpw,), jnp.int32),
            pltpu.VMEM((bpw, 128), jnp.float32),
            pltpu.SemaphoreType.DMA,
        ],
    )
    def gath(table_hbm, idx_hbm, out_hbm, idx_v, rows_v, sem):
        wid = lax.axis_index("s") * nc + lax.axis_index("c")
        base = wid * bpw
        pltpu.sync_copy(idx_hbm.at[pl.ds(base, bpw)], idx_v)
        pltpu.async_copy(table_hbm.at[idx_v], rows_v, sem).wait()
        pltpu.sync_copy(rows_v, out_hbm.at[pl.ds(base, bpw)])

    return gath


def _score_kernel(cls_ref, max_ref):
    max_ref[...] = jnp.max(cls_ref[...], axis=1, keepdims=True)


def _nms_kernel(vals_ref, pts_ref, bp_ref, ptsT_ref, bpT_ref, clsr_ref,
                clsrT_ref, ctr_ref, ctrT_ref, boxes_ref, scores_ref,
                keep_ref, cid_ref, mask_ref):
    joint = jax.nn.sigmoid(clsr_ref[...]) * jax.nn.sigmoid(ctr_ref[...])
    maxv = jnp.max(joint, axis=1, keepdims=True)
    ji = lax.broadcasted_iota(jnp.int32, joint.shape, 1)
    cid = jnp.min(jnp.where(joint == maxv, ji, NUM_CLASSES), axis=1,
                  keepdims=True)
    cid_ref[...] = cid

    jointT = jax.nn.sigmoid(clsrT_ref[...]) * jax.nn.sigmoid(ctrT_ref[...])
    maxvT = jnp.max(jointT, axis=0, keepdims=True)
    jiT = lax.broadcasted_iota(jnp.int32, jointT.shape, 0)
    cidT = jnp.min(jnp.where(jointT == maxvT, jiT, NUM_CLASSES), axis=0,
                   keepdims=True)

    x = pts_ref[:, 0:1]
    y = pts_ref[:, 1:2]
    l = bp_ref[:, 0:1] * FPN_STRIDE
    t = bp_ref[:, 1:2] * FPN_STRIDE
    r = bp_ref[:, 2:3] * FPN_STRIDE
    b = bp_ref[:, 3:4] * FPN_STRIDE
    x1 = jnp.clip(x - l, 0.0, IMG_W - 1.0)
    y1 = jnp.clip(y - t, 0.0, IMG_H - 1.0)
    x2 = jnp.clip(x + r, 0.0, IMG_W - 1.0)
    y2 = jnp.clip(y + b, 0.0, IMG_H - 1.0)
    off_c = cid.astype(jnp.float32) * (IMG_W + IMG_H + 1.0)
    x1c = x1 + off_c
    y1c = y1 + off_c
    x2c = x2 + off_c
    y2c = y2 + off_c
    area_c = jnp.clip(x2 - x1, 0.0, None) * jnp.clip(y2 - y1, 0.0, None)

    xr = ptsT_ref[0:1, :]
    yr = ptsT_ref[1:2, :]
    lr = bpT_ref[0:1, :] * FPN_STRIDE
    tr = bpT_ref[1:2, :] * FPN_STRIDE
    rr = bpT_ref[2:3, :] * FPN_STRIDE
    br = bpT_ref[3:4, :] * FPN_STRIDE
    x1r = jnp.clip(xr - lr, 0.0, IMG_W - 1.0)
    y1r = jnp.clip(yr - tr, 0.0, IMG_H - 1.0)
    x2r = jnp.clip(xr + rr, 0.0, IMG_W - 1.0)
    y2r = jnp.clip(yr + br, 0.0, IMG_H - 1.0)
    off_r = cidT.astype(jnp.float32) * (IMG_W + IMG_H + 1.0)
    x1rr = x1r + off_r
    y1rr = y1r + off_r
    x2rr = x2r + off_r
    y2rr = y2r + off_r
    area_r = jnp.clip(x2r - x1r, 0.0, None) * jnp.clip(y2r - y1r, 0.0, None)

    iw = jnp.clip(jnp.minimum(x2c, x2rr) - jnp.maximum(x1c, x1rr), 0.0, None)
    ih = jnp.clip(jnp.minimum(y2c, y2rr) - jnp.maximum(y1c, y1rr), 0.0, None)
    inter = iw * ih
    union = area_c + area_r - inter
    ii = lax.broadcasted_iota(jnp.int32, (M, 1), 0)
    jj = lax.broadcasted_iota(jnp.int32, (1, M), 1)
    tri = jnp.where(ii < jj, 1.0, 0.0)
    sup = inter > NMS_THRESHOLD * (union + 1e-9)
    mask_ref[...] = jnp.where(sup, tri, 0.0)

    valid = jnp.where(jj < NMS_PRE_TOPK, 1.0, 0.0)

    def cond(c):
        _, changed, it = c
        return changed & (it < NMS_PRE_TOPK)

    def body(c):
        k, _, it = c
        s = jnp.dot(k, mask_ref[...], preferred_element_type=jnp.float32)
        kn = jnp.where(s > 0.5, 0.0, valid)
        return kn, jnp.any(kn != k), it + 1

    keep, _, _ = lax.while_loop(cond, body, (valid, True, 0))
    keep_ref[...] = keep

    boxes_ref[...] = jnp.concatenate([x1, y1, x2, y2], axis=1)
    scores_ref[...] = jnp.where(ii < NMS_PRE_TOPK,
                                jnp.sqrt(jnp.abs(vals_ref[...])), -1.0)


def kernel(cls_scores, bbox_preds, centernesses, points):
    nblk = N_LOCS // SCORE_BLK
    maxv = pl.pallas_call(
        _score_kernel,
        grid=(nblk,),
        in_specs=[
            pl.BlockSpec((SCORE_BLK, NUM_CLASSES), lambda i: (i, 0)),
        ],
        out_specs=pl.BlockSpec((SCORE_BLK, 1), lambda i: (i, 0)),
        out_shape=jax.ShapeDtypeStruct((N_LOCS, 1), jnp.float32),
    )(cls_scores)
    max_scores = jax.nn.sigmoid(maxv.reshape(N_LOCS)) * jax.nn.sigmoid(
        centernesses)

    top_vals, top_inds = lax.top_k(max_scores, NMS_PRE_TOPK)
    pad = M - NMS_PRE_TOPK
    table = jnp.concatenate(
        [points, bbox_preds, centernesses[:, None], cls_scores,
         jnp.zeros((N_LOCS, 128 - 7 - NUM_CLASSES), jnp.float32)],
        axis=1)
    idx_p = jnp.pad(top_inds, (0, pad))
    rows_g = _make_sc_gather()(table, idx_p)
    pts_p = rows_g[:, 0:2]
    bp_p = rows_g[:, 2:6]
    ctr_p = rows_g[:, 6:7]
    clsr_p = rows_g[:, 7:7 + NUM_CLASSES]

    vals_p = jnp.pad(top_vals, (0, pad), constant_values=-1.0).reshape(M, 1)

    boxes, det_scores, keep, cid = pl.pallas_call(
        _nms_kernel,
        out_shape=(
            jax.ShapeDtypeStruct((M, 4), jnp.float32),
            jax.ShapeDtypeStruct((M, 1), jnp.float32),
            jax.ShapeDtypeStruct((1, M), jnp.float32),
            jax.ShapeDtypeStruct((M, 1), jnp.int32),
        ),
        scratch_shapes=[pltpu.VMEM((M, M), jnp.float32)],
    )(vals_p, pts_p, bp_p, pts_p.T, bp_p.T, clsr_p, clsr_p.T, ctr_p, ctr_p.T)

    boxes = boxes[:NMS_PRE_TOPK]
    det_scores = det_scores.reshape(M)[:NMS_PRE_TOPK]
    keep = keep.reshape(M)[:NMS_PRE_TOPK] > 0.5
    cls = cid.reshape(M)[:NMS_PRE_TOPK]

    kept_scores = jnp.where(keep, det_scores, -1.0)
    post_vals, post_inds = lax.top_k(kept_scores, NMS_POST_TOPK)
    out_boxes = jnp.take(boxes, post_inds, axis=0)
    out_classes = jnp.take(cls, post_inds)
    out = jnp.concatenate([out_boxes, post_vals[:, None]], axis=-1)
    return out, out_classes


# speedup vs baseline: 1.6702x; 1.6702x over previous
import functools

import jax
import jax.numpy as jnp
from jax import lax
from jax.experimental import pallas as pl
from jax.experimental.pallas import tpu as pltpu
from jax.experimental.pallas import tpu_sc as plsc

NUM_CLASSES = 80
FPN_STRIDE = 8.0
NMS_PRE_TOPK = 1000
NMS_THRESHOLD = 0.6
NMS_POST_TOPK = 100
IMG_H = 800
IMG_W = 1333
N_LOCS = 20000
M = 1024
SCORE_BLK = 4000


def _make_sc_gather():
    info = plsc.get_sparse_core_info()
    nc, ns = info.num_cores, info.num_subcores
    nw = nc * ns
    bpw = M // nw
    mesh = plsc.VectorSubcoreMesh(core_axis_name="c", subcore_axis_name="s")

    @functools.partial(
        pl.kernel, mesh=mesh,
        out_type=jax.ShapeDtypeStruct((M, 128), jnp.float32),
        scratch_types=[
            pltpu.VMEM((bpw,), jnp.int32),
            pltpu.VMEM((bpw, 128), jnp.float32),
            pltpu.SemaphoreType.DMA,
        ],
    )
    def gath(table_hbm, idx_hbm, out_hbm, idx_v, rows_v, sem):
        wid = lax.axis_index("s") * nc + lax.axis_index("c")
        base = wid * bpw
        pltpu.sync_copy(idx_hbm.at[pl.ds(base, bpw)], idx_v)
        pltpu.async_copy(table_hbm.at[idx_v], rows_v, sem).wait()
        pltpu.sync_copy(rows_v, out_hbm.at[pl.ds(base, bpw)])

    return gath


def _score_kernel(cls_ref, max_ref):
    max_ref[...] = jnp.max(cls_ref[...], axis=1, keepdims=True)


def _nms_kernel(vals_ref, pts_ref, bp_ref, ptsT_ref, bpT_ref, clsr_ref,
                clsrT_ref, ctr_ref, ctrT_ref, boxes_ref, scores_ref,
                keep_ref, cid_ref, mask_ref):
    joint = jax.nn.sigmoid(clsr_ref[...]) * jax.nn.sigmoid(ctr_ref[...])
    maxv = jnp.max(joint, axis=1, keepdims=True)
    ji = lax.broadcasted_iota(jnp.int32, joint.shape, 1)
    cid = jnp.min(jnp.where(joint == maxv, ji, NUM_CLASSES), axis=1,
                  keepdims=True)
    cid_ref[...] = cid

    jointT = jax.nn.sigmoid(clsrT_ref[...]) * jax.nn.sigmoid(ctrT_ref[...])
    maxvT = jnp.max(jointT, axis=0, keepdims=True)
    jiT = lax.broadcasted_iota(jnp.int32, jointT.shape, 0)
    cidT = jnp.min(jnp.where(jointT == maxvT, jiT, NUM_CLASSES), axis=0,
                   keepdims=True)

    x = pts_ref[:, 0:1]
    y = pts_ref[:, 1:2]
    l = bp_ref[:, 0:1] * FPN_STRIDE
    t = bp_ref[:, 1:2] * FPN_STRIDE
    r = bp_ref[:, 2:3] * FPN_STRIDE
    b = bp_ref[:, 3:4] * FPN_STRIDE
    x1 = jnp.clip(x - l, 0.0, IMG_W - 1.0)
    y1 = jnp.clip(y - t, 0.0, IMG_H - 1.0)
    x2 = jnp.clip(x + r, 0.0, IMG_W - 1.0)
    y2 = jnp.clip(y + b, 0.0, IMG_H - 1.0)
    off_c = cid.astype(jnp.float32) * (IMG_W + IMG_H + 1.0)
    x1c = x1 + off_c
    y1c = y1 + off_c
    x2c = x2 + off_c
    y2c = y2 + off_c
    area_c = jnp.clip(x2 - x1, 0.0, None) * jnp.clip(y2 - y1, 0.0, None)

    xr = ptsT_ref[0:1, :]
    yr = ptsT_ref[1:2, :]
    lr = bpT_ref[0:1, :] * FPN_STRIDE
    tr = bpT_ref[1:2, :] * FPN_STRIDE
    rr = bpT_ref[2:3, :] * FPN_STRIDE
    br = bpT_ref[3:4, :] * FPN_STRIDE
    x1r = jnp.clip(xr - lr, 0.0, IMG_W - 1.0)
    y1r = jnp.clip(yr - tr, 0.0, IMG_H - 1.0)
    x2r = jnp.clip(xr + rr, 0.0, IMG_W - 1.0)
    y2r = jnp.clip(yr + br, 0.0, IMG_H - 1.0)
    off_r = cidT.astype(jnp.float32) * (IMG_W + IMG_H + 1.0)
    x1rr = x1r + off_r
    y1rr = y1r + off_r
    x2rr = x2r + off_r
    y2rr = y2r + off_r
    area_r = jnp.clip(x2r - x1r, 0.0, None) * jnp.clip(y2r - y1r, 0.0, None)

    iw = jnp.clip(jnp.minimum(x2c, x2rr) - jnp.maximum(x1c, x1rr), 0.0, None)
    ih = jnp.clip(jnp.minimum(y2c, y2rr) - jnp.maximum(y1c, y1rr), 0.0, None)
    inter = iw * ih
    union = area_c + area_r - inter
    ii = lax.broadcasted_iota(jnp.int32, (M, 1), 0)
    jj = lax.broadcasted_iota(jnp.int32, (1, M), 1)
    tri = jnp.where(ii < jj, 1.0, 0.0)
    sup = inter > NMS_THRESHOLD * (union + 1e-9)
    mask_ref[...] = jnp.where(sup, tri, 0.0)

    valid = jnp.where(jj < NMS_PRE_TOPK, 1.0, 0.0)

    def cond(c):
        _, changed, it = c
        return changed & (it < NMS_PRE_TOPK)

    def body(c):
        k, _, it = c
        s = jnp.dot(k, mask_ref[...], preferred_element_type=jnp.float32)
        kn = jnp.where(s > 0.5, 0.0, valid)
        return kn, jnp.any(kn != k), it + 1

    keep, _, _ = lax.while_loop(cond, body, (valid, True, 0))
    keep_ref[...] = keep

    boxes_ref[...] = jnp.concatenate([x1, y1, x2, y2], axis=1)
    scores_ref[...] = jnp.where(ii < NMS_PRE_TOPK,
                                jnp.sqrt(jnp.abs(vals_ref[...])), -1.0)


def kernel(cls_scores, bbox_preds, centernesses, points):
    nblk = N_LOCS // SCORE_BLK
    maxv = pl.pallas_call(
        _score_kernel,
        grid=(nblk,),
        in_specs=[
            pl.BlockSpec((SCORE_BLK, NUM_CLASSES), lambda i: (i, 0)),
        ],
        out_specs=pl.BlockSpec((SCORE_BLK, 1), lambda i: (i, 0)),
        out_shape=jax.ShapeDtypeStruct((N_LOCS, 1), jnp.float32),
    )(cls_scores)
    max_scores = jax.nn.sigmoid(maxv.reshape(N_LOCS)) * jax.nn.sigmoid(
        centernesses)

    top_vals, top_inds = lax.top_k(max_scores, NMS_PRE_TOPK)
    pad = M - NMS_PRE_TOPK
    small = jnp.concatenate(
        [points, bbox_preds, centernesses[:, None],
         jnp.zeros((N_LOCS, 1), jnp.float32)], axis=1)
    idx_p = jnp.pad(top_inds, (0, pad))
    small_g = jnp.take(small, idx_p, axis=0)
    clsr_p = jnp.take(cls_scores, idx_p, axis=0)
    pts_p = small_g[:, 0:2]
    bp_p = small_g[:, 2:6]
    ctr_p = small_g[:, 6:7]

    vals_p = jnp.pad(top_vals, (0, pad), constant_values=-1.0).reshape(M, 1)

    boxes, det_scores, keep, cid = pl.pallas_call(
        _nms_kernel,
        out_shape=(
            jax.ShapeDtypeStruct((M, 4), jnp.float32),
            jax.ShapeDtypeStruct((M, 1), jnp.float32),
            jax.ShapeDtypeStruct((1, M), jnp.float32),
            jax.ShapeDtypeStruct((M, 1), jnp.int32),
        ),
        scratch_shapes=[pltpu.VMEM((M, M), jnp.float32)],
    )(vals_p, pts_p, bp_p, pts_p.T, bp_p.T, clsr_p, clsr_p.T, ctr_p, ctr_p.T)

    boxes = boxes[:NMS_PRE_TOPK]
    det_scores = det_scores.reshape(M)[:NMS_PRE_TOPK]
    keep = keep.reshape(M)[:NMS_PRE_TOPK] > 0.5
    cls = cid.reshape(M)[:NMS_PRE_TOPK]

    kept_scores = jnp.where(keep, det_scores, -1.0)
    post_vals, post_inds = lax.top_k(kept_scores, NMS_POST_TOPK)
    out_boxes = jnp.take(boxes, post_inds, axis=0)
    out_classes = jnp.take(cls, post_inds)
    out = jnp.concatenate([out_boxes, post_vals[:, None]], axis=-1)
    return out, out_classes
